# Initial kernel scaffold; baseline (speedup 1.0000x reference)
import jax
import jax.numpy as jnp
from jax.experimental import pallas as pl


def kernel(x, apply_indices, pos_embedding):
    raise NotImplementedError("write your pallas kernel here")


# speedup vs baseline: 1.3772x; 1.3772x over previous
import functools

import jax
import jax.numpy as jnp
from jax import lax
from jax.experimental import pallas as pl
from jax.experimental.pallas import tpu as pltpu
from jax.experimental.pallas import tpu_sc as plsc

B = 4
N = 8192
EMB = 768
ROWS = B * N
NC = 2
NS = 16
NW = NC * NS
RPW = ROWS // NW
K = 64
NCHUNK = RPW // K
LANES = 16
CPV = EMB // LANES

_mesh = plsc.VectorSubcoreMesh(core_axis_name="c", subcore_axis_name="s")


@functools.partial(
    pl.kernel,
    mesh=_mesh,
    out_type=jax.ShapeDtypeStruct((ROWS, EMB), jnp.float32),
    scratch_types=[
        pltpu.VMEM((RPW,), jnp.int32),
        pltpu.VMEM((K, EMB), jnp.float32),
        pltpu.VMEM((K, EMB), jnp.float32),
        pltpu.SemaphoreType.DMA,
    ],
)
def _pos_emb_sc(x_hbm, idx_hbm, tab_hbm, out_hbm, idx_v, gb, xb, insem):
    wid = lax.axis_index("s") * NC + lax.axis_index("c")
    base = wid * RPW
    pltpu.sync_copy(idx_hbm.at[pl.ds(base, RPW)], idx_v)

    def chunk_body(g, carry):
        row0 = base + g * K
        hg = pltpu.async_copy(tab_hbm.at[idx_v.at[pl.ds(g * K, K)]], gb, insem)
        hx = pltpu.async_copy(x_hbm.at[pl.ds(row0, K)], xb, insem)
        hg.wait()
        hx.wait()

        def row_body(r, c2):
            for c in range(CPV):
                sl = pl.ds(c * LANES, LANES)
                xb[r, sl] += gb[r, sl]
            return c2

        lax.fori_loop(0, K, row_body, 0)
        pltpu.sync_copy(xb, out_hbm.at[pl.ds(row0, K)])
        return carry

    lax.fori_loop(0, NCHUNK, chunk_body, 0)


def kernel(x, apply_indices, pos_embedding):
    xf = x.reshape(ROWS, EMB)
    idx = apply_indices.reshape(ROWS).astype(jnp.int32)
    out = _pos_emb_sc(xf, idx, pos_embedding)
    return out.reshape(x.shape)


# speedup vs baseline: 2.0368x; 1.4790x over previous
import functools

import jax
import jax.numpy as jnp
from jax import lax
from jax.experimental import pallas as pl
from jax.experimental.pallas import tpu as pltpu
from jax.experimental.pallas import tpu_sc as plsc

B = 4
N = 8192
EMB = 768
ROWS = B * N
NC = 2
NS = 16
NW = NC * NS
RPW = ROWS // NW
K = 32
NCHUNK = RPW // K
NPAIR = NCHUNK // 2
LANES = 16
CPV = EMB // LANES

_mesh = plsc.VectorSubcoreMesh(core_axis_name="c", subcore_axis_name="s")


@functools.partial(
    pl.kernel,
    mesh=_mesh,
    out_type=jax.ShapeDtypeStruct((ROWS, EMB), jnp.float32),
    scratch_types=[
        pltpu.VMEM((RPW,), jnp.int32),
        pltpu.VMEM((K, EMB), jnp.float32),
        pltpu.VMEM((K, EMB), jnp.float32),
        pltpu.VMEM((K, EMB), jnp.float32),
        pltpu.VMEM((K, EMB), jnp.float32),
        pltpu.SemaphoreType.DMA,
        pltpu.SemaphoreType.DMA,
        pltpu.SemaphoreType.DMA,
        pltpu.SemaphoreType.DMA,
    ],
)
def _pos_emb_sc(x_hbm, idx_hbm, tab_hbm, out_hbm,
                idx_v, g0, g1, x0, x1, in0, in1, o0, o1):
    wid = lax.axis_index("s") * NC + lax.axis_index("c")
    base = wid * RPW
    pltpu.sync_copy(idx_hbm.at[pl.ds(base, RPW)], idx_v)

    def start_loads(g, gb, xb, sem):
        pltpu.async_copy(tab_hbm.at[idx_v.at[pl.ds(g * K, K)]], gb, sem)
        pltpu.async_copy(x_hbm.at[pl.ds(base + g * K, K)], xb, sem)

    def wait_loads(gb, xb, sem):
        pltpu.make_async_copy(tab_hbm.at[idx_v.at[pl.ds(0, K)]], gb, sem).wait()
        pltpu.make_async_copy(x_hbm.at[pl.ds(base, K)], xb, sem).wait()

    def wait_out(xb, sem):
        pltpu.make_async_copy(xb, out_hbm.at[pl.ds(base, K)], sem).wait()

    def compute(gb, xb):
        def row_body(r, carry):
            for c in range(CPV):
                sl = pl.ds(c * LANES, LANES)
                plsc.addupdate(xb.at[r, sl], gb[r, sl])
            return carry
        lax.fori_loop(0, K, row_body, 0)

    start_loads(0, g0, x0, in0)

    def pair_body(i, carry):
        a = 2 * i

        @pl.when(i > 0)
        def _():
            wait_out(x1, o1)
        start_loads(a + 1, g1, x1, in1)
        wait_loads(g0, x0, in0)
        compute(g0, x0)
        pltpu.async_copy(x0, out_hbm.at[pl.ds(base + a * K, K)], o0)

        @pl.when(i < NPAIR - 1)
        def _():
            wait_out(x0, o0)
            start_loads(a + 2, g0, x0, in0)
        wait_loads(g1, x1, in1)
        compute(g1, x1)
        pltpu.async_copy(x1, out_hbm.at[pl.ds(base + (a + 1) * K, K)], o1)
        return carry

    lax.fori_loop(0, NPAIR, pair_body, 0)
    wait_out(x0, o0)
    wait_out(x1, o1)


def kernel(x, apply_indices, pos_embedding):
    xf = x.reshape(ROWS, EMB)
    idx = apply_indices.reshape(ROWS).astype(jnp.int32)
    out = _pos_emb_sc(xf, idx, pos_embedding)
    return out.reshape(x.shape)
